# baseline (speedup 1.0000x reference)
import functools

import jax
import jax.numpy as jnp
from jax.experimental import pallas as pl
from jax.experimental.pallas import tpu as pltpu


def _roi_body(fm_ref, cd_ref, sc_ref, out_ref, *, H, W):
    N = sc_ref.shape[2]
    ch = fm_ref.shape[1]
    cd = cd_ref[0]
    x0 = cd[0:1, :]
    x1 = cd[1:2, :]
    y0 = cd[2:3, :]
    y1 = cd[3:4, :]

    xi = jax.lax.broadcasted_iota(jnp.int32, (W, N), 0)
    colt = jnp.where((xi >= x0) & (xi < x1), 1.0, 0.0).astype(jnp.float32)

    wu = pltpu.bitcast(fm_ref[0], jnp.uint32)
    acc_lo = jnp.zeros((ch, N), jnp.float32)
    acc_hi = jnp.zeros((ch, N), jnp.float32)
    for yc in range(0, H, 8):
        rows = min(8, H - yc)
        wc = wu[:, yc:yc + rows, :].reshape(ch * rows, W)
        xlo = pltpu.bitcast(wc << 16, jnp.float32)
        xhi = pltpu.bitcast(wc & jnp.uint32(0xFFFF0000), jnp.float32)
        ulo = jnp.dot(xlo, colt, preferred_element_type=jnp.float32)
        uhi = jnp.dot(xhi, colt, preferred_element_type=jnp.float32)
        ulo = ulo.reshape(ch, rows, N)
        uhi = uhi.reshape(ch, rows, N)
        yi = jax.lax.broadcasted_iota(jnp.int32, (rows, N), 0) + yc
        rc = jnp.where((yi >= y0) & (yi < y1), 1.0, 0.0).astype(jnp.float32)
        acc_lo = acc_lo + jnp.sum(ulo * rc[None, :, :], axis=1)
        acc_hi = acc_hi + jnp.sum(uhi * rc[None, :, :], axis=1)
    sc = sc_ref[0]
    out_ref[0, :ch, :] = acc_lo * sc
    out_ref[0, ch:, :] = acc_hi * sc


def kernel(feature_map, keypoints, mask, original_H, original_W):
    B, C, H, W = feature_map.shape
    N = keypoints.shape[1]
    sx = W / original_W
    sy = H / original_H
    x, y, w, h = (keypoints[..., 0], keypoints[..., 1],
                  keypoints[..., 2], keypoints[..., 3])
    xr = jnp.clip((x * sx).astype(jnp.int32), 0, W - 1)
    yr = jnp.clip((y * sy).astype(jnp.int32), 0, H - 1)
    wr = jnp.minimum(jnp.maximum((w * sx).astype(jnp.int32), 1), W - xr)
    hr = jnp.minimum(jnp.maximum((h * sy).astype(jnp.int32), 1), H - yr)
    coords = jnp.stack([xr, xr + wr, yr, yr + hr], axis=1)
    area = (hr * wr).astype(jnp.float32)
    scale = jnp.where(mask > 0, 1.0 / area, 0.0).reshape(B, 1, N)

    def _rne(u):
        return (u + jnp.uint32(0x7FFF) + ((u >> 16) & jnp.uint32(1))) >> 16

    ch = C // 2
    u5 = jax.lax.bitcast_convert_type(
        feature_map.reshape(B, 2, ch, H, W), jnp.uint32)
    shifts = jnp.array([0, 16], jnp.uint32).reshape(1, 2, 1, 1, 1)
    packed = jax.lax.bitcast_convert_type(
        jnp.sum(_rne(u5) << shifts, axis=1, dtype=jnp.uint32), jnp.int32)

    out = pl.pallas_call(
        functools.partial(_roi_body, H=H, W=W),
        grid=(B,),
        in_specs=[
            pl.BlockSpec((1, ch, H, W), lambda b: (b, 0, 0, 0)),
            pl.BlockSpec((1, 4, N), lambda b: (b, 0, 0)),
            pl.BlockSpec((1, 1, N), lambda b: (b, 0, 0)),
        ],
        out_specs=pl.BlockSpec((1, C, N), lambda b: (b, 0, 0)),
        out_shape=jax.ShapeDtypeStruct((B, C, N), jnp.float32),
        compiler_params=pltpu.CompilerParams(
            dimension_semantics=("parallel",),
            vmem_limit_bytes=50 * 1024 * 1024,
        ),
    )(packed, coords, scale)
    return jnp.transpose(out, (0, 2, 1))


# speedup vs baseline: 3.5508x; 3.5508x over previous
import functools

import jax
import jax.numpy as jnp
from jax.experimental import pallas as pl
from jax.experimental.pallas import tpu as pltpu


def _roi_body(fm_ref, cd_ref, sc_ref, out_ref, *, H, W):
    N = sc_ref.shape[2]
    cd = cd_ref[0]
    x0 = cd[0:1, :]
    x1 = cd[1:2, :]
    y0 = cd[2:3, :]
    y1 = cd[3:4, :]

    xi = jax.lax.broadcasted_iota(jnp.int32, (W, N), 0)
    colf = jnp.where((xi >= x0) & (xi < x1), 1.0, 0.0).astype(jnp.float32)
    yi = jax.lax.broadcasted_iota(jnp.int32, (H, N), 0)
    rowf = jnp.where((yi >= y0) & (yi < y1), 1.0, 0.0).astype(jnp.float32)

    m3 = rowf[:, None, :] * colf[None, :, :]
    ind = m3.reshape(H * W, N).astype(jnp.bfloat16)

    acc = jnp.dot(fm_ref[0], ind, preferred_element_type=jnp.float32)
    out_ref[0] = acc * sc_ref[0]


def kernel(feature_map, keypoints, mask, original_H, original_W):
    B, C, H, W = feature_map.shape
    N = keypoints.shape[1]
    sx = W / original_W
    sy = H / original_H
    x, y, w, h = (keypoints[..., 0], keypoints[..., 1],
                  keypoints[..., 2], keypoints[..., 3])
    xr = jnp.clip((x * sx).astype(jnp.int32), 0, W - 1)
    yr = jnp.clip((y * sy).astype(jnp.int32), 0, H - 1)
    wr = jnp.minimum(jnp.maximum((w * sx).astype(jnp.int32), 1), W - xr)
    hr = jnp.minimum(jnp.maximum((h * sy).astype(jnp.int32), 1), H - yr)
    coords = jnp.stack([xr, xr + wr, yr, yr + hr], axis=1)
    area = (hr * wr).astype(jnp.float32)
    scale = jnp.where(mask > 0, 1.0 / area, 0.0).reshape(B, 1, N)

    fm = feature_map.reshape(B, C, H * W).astype(jnp.bfloat16)
    c_blk = 128
    grid = (B, C // c_blk)
    out = pl.pallas_call(
        functools.partial(_roi_body, H=H, W=W),
        grid=grid,
        in_specs=[
            pl.BlockSpec((1, c_blk, H * W), lambda b, c: (b, c, 0)),
            pl.BlockSpec((1, 4, N), lambda b, c: (b, 0, 0)),
            pl.BlockSpec((1, 1, N), lambda b, c: (b, 0, 0)),
        ],
        out_specs=pl.BlockSpec((1, c_blk, N), lambda b, c: (b, c, 0)),
        out_shape=jax.ShapeDtypeStruct((B, C, N), jnp.float32),
        compiler_params=pltpu.CompilerParams(
            dimension_semantics=("parallel", "arbitrary"),
            vmem_limit_bytes=50 * 1024 * 1024,
        ),
    )(fm, coords, scale)
    return jnp.transpose(out, (0, 2, 1))


# speedup vs baseline: 3.6599x; 1.0307x over previous
import functools

import jax
import jax.numpy as jnp
from jax.experimental import pallas as pl
from jax.experimental.pallas import tpu as pltpu


def _roi_body(fm_ref, cd_ref, sc_ref, out_ref, *, H, W):
    N = sc_ref.shape[2]
    cd = cd_ref[0]
    x0 = cd[0:1, :]
    x1 = cd[1:2, :]
    y0 = cd[2:3, :]
    y1 = cd[3:4, :]

    xi = jax.lax.broadcasted_iota(jnp.int32, (W, N), 0)
    colf = jnp.where((xi >= x0) & (xi < x1), 1.0, 0.0).astype(jnp.float32)
    yi = jax.lax.broadcasted_iota(jnp.int32, (H, N), 0)
    rowf = jnp.where((yi >= y0) & (yi < y1), 1.0, 0.0).astype(jnp.float32)

    m3 = rowf[:, None, :] * colf[None, :, :]
    ind = m3.reshape(H * W, N).astype(jnp.bfloat16)

    acc = jnp.dot(fm_ref[0], ind, preferred_element_type=jnp.float32)
    out_ref[0] = acc * sc_ref[0]


def kernel(feature_map, keypoints, mask, original_H, original_W):
    B, C, H, W = feature_map.shape
    N = keypoints.shape[1]
    sx = W / original_W
    sy = H / original_H
    x, y, w, h = (keypoints[..., 0], keypoints[..., 1],
                  keypoints[..., 2], keypoints[..., 3])
    xr = jnp.clip((x * sx).astype(jnp.int32), 0, W - 1)
    yr = jnp.clip((y * sy).astype(jnp.int32), 0, H - 1)
    wr = jnp.minimum(jnp.maximum((w * sx).astype(jnp.int32), 1), W - xr)
    hr = jnp.minimum(jnp.maximum((h * sy).astype(jnp.int32), 1), H - yr)
    coords = jnp.stack([xr, xr + wr, yr, yr + hr], axis=1)
    area = (hr * wr).astype(jnp.float32)
    scale = jnp.where(mask > 0, 1.0 / area, 0.0).reshape(B, 1, N)

    fm = feature_map.reshape(B, C, H * W).astype(jnp.bfloat16)
    out = pl.pallas_call(
        functools.partial(_roi_body, H=H, W=W),
        grid=(B,),
        in_specs=[
            pl.BlockSpec((1, C, H * W), lambda b: (b, 0, 0)),
            pl.BlockSpec((1, 4, N), lambda b: (b, 0, 0)),
            pl.BlockSpec((1, 1, N), lambda b: (b, 0, 0)),
        ],
        out_specs=pl.BlockSpec((1, C, N), lambda b: (b, 0, 0)),
        out_shape=jax.ShapeDtypeStruct((B, C, N), jnp.float32),
        compiler_params=pltpu.CompilerParams(
            dimension_semantics=("parallel",),
            vmem_limit_bytes=50 * 1024 * 1024,
        ),
    )(fm, coords, scale)
    return jnp.transpose(out, (0, 2, 1))
